# baseline (speedup 1.0000x reference)
import functools

import jax
import jax.numpy as jnp
from jax import lax
from jax.experimental import pallas as pl
from jax.experimental.pallas import tpu as pltpu
from jax.experimental.pallas import tpu_sc as plsc

N = 10000
D = 128
NC = 2
NS = 16
NW = NC * NS
WE = 128
NACC = 10240
PTN = NACC // NS
NPAD_ROWS = NACC - N
NBUF = 2
NIB = 4

_mesh = plsc.VectorSubcoreMesh(core_axis_name="c", subcore_axis_name="s")


def _nwin(E):
    n = (E + NW * WE - 1) // (NW * WE)
    return ((n + NIB - 1) // NIB) * NIB


def _make_deg(nwin):
    @functools.partial(
        pl.kernel,
        out_type=jax.ShapeDtypeStruct((NC, NACC), jnp.float32),
        mesh=_mesh,
        scratch_types=[
            pltpu.VMEM((nwin, 2, WE), jnp.int32),
            pltpu.VMEM((PTN,), jnp.float32),
            pltpu.VMEM((WE,), jnp.float32),
            pltpu.VMEM_SHARED((NACC,), jnp.float32),
        ],
    )
    def deg_kernel(idx_hbm, deg_out, idx_v, zv, ones_v, acc):
        c = lax.axis_index("c")
        s = lax.axis_index("s")
        w = c * NS + s

        def fz(i, _):
            zv[pl.ds(i * 16, 16)] = jnp.zeros((16,), jnp.float32)
            return _
        lax.fori_loop(0, PTN // 16, fz, None)

        def fo(i, _):
            ones_v[pl.ds(i * 16, 16)] = jnp.ones((16,), jnp.float32)
            return _
        lax.fori_loop(0, WE // 16, fo, None)

        pltpu.sync_copy(idx_hbm.at[w], idx_v)
        pltpu.sync_copy(zv, acc.at[pl.ds(s * PTN, PTN)])
        plsc.subcore_barrier()

        def body(j, _):
            pltpu.sync_copy(ones_v, acc.at[idx_v.at[j, 1]], add=True)
            return _
        lax.fori_loop(0, nwin, body, None)

        plsc.subcore_barrier()
        pltpu.sync_copy(acc.at[pl.ds(s * PTN, PTN)],
                        deg_out.at[c, pl.ds(s * PTN, PTN)])

    return deg_kernel


def _make_agg(nwin):
    assert nwin % NIB == 0 and nwin >= 2 * NIB

    @functools.partial(
        pl.kernel,
        out_type=jax.ShapeDtypeStruct((NC, NACC, D), jnp.float32),
        mesh=_mesh,
        scratch_types=[
            pltpu.VMEM((NIB, 2, WE), jnp.int32),
            pltpu.VMEM((NBUF, WE, D), jnp.float32),
            pltpu.VMEM((16, D), jnp.float32),
            pltpu.VMEM_SHARED((NACC, D), jnp.float32),
            [pltpu.SemaphoreType.DMA] * NIB,
            [pltpu.SemaphoreType.DMA] * NBUF,
            [pltpu.SemaphoreType.DMA] * NBUF,
        ],
    )
    def agg_kernel(m_hbm, idx_hbm, g_out,
                   iring, buf, zb, acc, isems, gsems, ssems):
        c = lax.axis_index("c")
        s = lax.axis_index("s")
        w = c * NS + s

        def fz(i, _):
            zb[i // 8, pl.ds((i % 8) * 16, 16)] = jnp.zeros((16,), jnp.float32)
            return _
        lax.fori_loop(0, 16 * 8, fz, None)

        base = s * PTN

        def zacc(k, _):
            pltpu.sync_copy(zb, acc.at[pl.ds(base + k * 16, 16)])
            return _
        lax.fori_loop(0, PTN // 16, zacc, None)
        plsc.subcore_barrier()

        def i_start(j, ib):
            pltpu.async_copy(idx_hbm.at[w, j], iring.at[ib], isems[ib])

        def i_wait(j, ib):
            pltpu.make_async_copy(idx_hbm.at[w, j], iring.at[ib],
                                  isems[ib]).wait()

        def g_start(j, ib, b):
            pltpu.async_copy(m_hbm.at[iring.at[ib, 0]], buf.at[b], gsems[b])

        def g_wait(j, ib, b):
            pltpu.make_async_copy(m_hbm.at[iring.at[ib, 0]], buf.at[b],
                                  gsems[b]).wait()

        def s_start(j, ib, b):
            pltpu.async_copy(buf.at[b], acc.at[iring.at[ib, 1]], ssems[b],
                             add=True)

        def s_wait(j, ib, b):
            pltpu.make_async_copy(buf.at[b], acc.at[iring.at[ib, 1]],
                                  ssems[b]).wait()

        i_start(0, 0)
        i_start(1, 1)
        i_wait(0, 0)
        g_start(0, 0, 0)
        i_wait(1, 1)
        g_start(1, 1, 1)
        i_start(2, 2)
        g_wait(0, 0, 0)
        s_start(0, 0, 0)
        s_wait(0, 0, 0)
        i_wait(2, 2)
        g_start(2, 2, 0)
        i_start(3, 3)
        g_wait(1, 1, 1)
        s_start(1, 1, 1)

        def body(t, _):
            j0 = 2 + t * NIB
            for k in range(NIB):
                j = j0 + k
                b = (2 + k) % NBUF
                ib = (2 + k) % NIB
                s_wait(j - 1, (ib - 1) % NIB, 1 - b)
                i_wait(j + 1, (ib + 1) % NIB)
                g_start(j + 1, (ib + 1) % NIB, 1 - b)
                i_start(j + 2, (ib + 2) % NIB)
                g_wait(j, ib, b)
                s_start(j, ib, b)
            return _
        lax.fori_loop(0, (nwin - 4) // NIB, body, None)

        j = nwin - 2
        b, ib = j % NBUF, j % NIB
        s_wait(j - 1, (ib - 1) % NIB, 1 - b)
        i_wait(j + 1, (ib + 1) % NIB)
        g_start(j + 1, (ib + 1) % NIB, 1 - b)
        g_wait(j, ib, b)
        s_start(j, ib, b)
        j = nwin - 1
        b, ib = j % NBUF, j % NIB
        s_wait(j - 1, (ib - 1) % NIB, 1 - b)
        g_wait(j, ib, b)
        s_start(j, ib, b)
        s_wait(j, ib, b)

        plsc.subcore_barrier()
        pltpu.sync_copy(acc.at[pl.ds(base, PTN)],
                        g_out.at[c, pl.ds(base, PTN)])

    return agg_kernel


BR = 400


def _prep_body(x_ref, w_ref, deg_ref, m_ref, dinv_ref):
    deg = deg_ref[0] + deg_ref[1]
    dv = jnp.where(deg > 0.0, lax.rsqrt(jnp.maximum(deg, 1e-12)), 0.0)
    dinv_ref[...] = dv
    h = jax.lax.dot(x_ref[...], w_ref[...],
                    precision=jax.lax.Precision.HIGHEST)
    m_ref[...] = h * dv


def _prep(x, W1, deg2):
    grid = N // BR
    return pl.pallas_call(
        _prep_body,
        grid=(grid,),
        in_specs=[
            pl.BlockSpec((BR, D), lambda i: (i, 0)),
            pl.BlockSpec((D, D), lambda i: (0, 0)),
            pl.BlockSpec((NC, BR, 1), lambda i: (0, i, 0)),
        ],
        out_specs=[
            pl.BlockSpec((BR, D), lambda i: (i, 0)),
            pl.BlockSpec((BR, 1), lambda i: (i, 0)),
        ],
        out_shape=[
            jax.ShapeDtypeStruct((N, D), jnp.float32),
            jax.ShapeDtypeStruct((N, 1), jnp.float32),
        ],
    )(x, W1, deg2)


def _mid_body(g_ref, dinv_ref, b_ref, w_ref, m_ref):
    dv = dinv_ref[...]
    agg = (g_ref[0] + g_ref[1]) * dv + b_ref[...]
    z = jnp.maximum(agg, 0.0)
    h = jax.lax.dot(z, w_ref[...], precision=jax.lax.Precision.HIGHEST)
    m_ref[...] = h * dv


def _mid(g, dinv, b, W):
    grid = N // BR
    return pl.pallas_call(
        _mid_body,
        grid=(grid,),
        in_specs=[
            pl.BlockSpec((NC, BR, D), lambda i: (0, i, 0)),
            pl.BlockSpec((BR, 1), lambda i: (i, 0)),
            pl.BlockSpec((1, D), lambda i: (0, 0)),
            pl.BlockSpec((D, D), lambda i: (0, 0)),
        ],
        out_specs=pl.BlockSpec((BR, D), lambda i: (i, 0)),
        out_shape=jax.ShapeDtypeStruct((N, D), jnp.float32),
    )(g, dinv, b.reshape(1, D), W)


def _final_body(g_ref, dinv_ref, b_ref, o_ref):
    dv = dinv_ref[...]
    o_ref[...] = (g_ref[0] + g_ref[1]) * dv + b_ref[...]


def _final(g, dinv, b):
    grid = N // BR
    return pl.pallas_call(
        _final_body,
        grid=(grid,),
        in_specs=[
            pl.BlockSpec((NC, BR, D), lambda i: (0, i, 0)),
            pl.BlockSpec((BR, 1), lambda i: (i, 0)),
            pl.BlockSpec((1, D), lambda i: (0, 0)),
        ],
        out_specs=pl.BlockSpec((BR, D), lambda i: (i, 0)),
        out_shape=jax.ShapeDtypeStruct((N, D), jnp.float32),
    )(g, dinv, b.reshape(1, D))


def kernel(x, adj_t, W1, b1, W2, b2, W3, b3):
    adj = adj_t.astype(jnp.int32)
    E = adj.shape[1]
    nwin = _nwin(E)
    epad = NW * WE * nwin
    pad = epad - E
    src = jnp.concatenate([adj[0], jnp.zeros((pad,), jnp.int32)])
    dst = jnp.concatenate(
        [adj[1], N + (jnp.arange(pad, dtype=jnp.int32) % NPAD_ROWS)])
    idx_w = jnp.stack(
        [src.reshape(NW, nwin, WE), dst.reshape(NW, nwin, WE)], axis=2)

    deg2 = _make_deg(nwin)(idx_w)
    agg = _make_agg(nwin)
    m1, dinv = _prep(x, W1, deg2.reshape(NC, NACC, 1))
    g1 = agg(m1, idx_w)
    m2 = _mid(g1, dinv, b1, W2)
    g2 = agg(m2, idx_w)
    m3 = _mid(g2, dinv, b2, W3)
    g3 = agg(m3, idx_w)
    return _final(g3, dinv, b3)


# speedup vs baseline: 3.4357x; 3.4357x over previous
import functools

import jax
import jax.numpy as jnp
from jax import lax
from jax.experimental import pallas as pl
from jax.experimental.pallas import tpu as pltpu
from jax.experimental.pallas import tpu_sc as plsc

N = 10000
D = 128
NC = 2
NS = 16
NW = NC * NS
WE = 128
NACC = 10240
PTN = NACC // NS
NPAD_ROWS = NACC - N
NBUF = 2
NIB = 4

_mesh = plsc.VectorSubcoreMesh(core_axis_name="c", subcore_axis_name="s")


def _nwin(E):
    n = (E + NW * WE - 1) // (NW * WE)
    return ((n + NIB - 1) // NIB) * NIB


def _make_deg(nwin):
    @functools.partial(
        pl.kernel,
        out_type=jax.ShapeDtypeStruct((NC, NACC), jnp.float32),
        mesh=_mesh,
        scratch_types=[
            pltpu.VMEM((nwin, 2, WE), jnp.int32),
            pltpu.VMEM((PTN,), jnp.float32),
            pltpu.VMEM((WE,), jnp.float32),
            pltpu.VMEM_SHARED((NACC,), jnp.float32),
        ],
    )
    def deg_kernel(idx_hbm, deg_out, idx_v, zv, ones_v, acc):
        c = lax.axis_index("c")
        s = lax.axis_index("s")
        w = c * NS + s

        def fz(i, _):
            zv[pl.ds(i * 16, 16)] = jnp.zeros((16,), jnp.float32)
            return _
        lax.fori_loop(0, PTN // 16, fz, None)

        def fo(i, _):
            ones_v[pl.ds(i * 16, 16)] = jnp.ones((16,), jnp.float32)
            return _
        lax.fori_loop(0, WE // 16, fo, None)

        pltpu.sync_copy(idx_hbm.at[w], idx_v)
        pltpu.sync_copy(zv, acc.at[pl.ds(s * PTN, PTN)])
        plsc.subcore_barrier()

        def body(j, _):
            pltpu.sync_copy(ones_v, acc.at[idx_v.at[j, 1]], add=True)
            return _
        lax.fori_loop(0, nwin, body, None)

        plsc.subcore_barrier()
        pltpu.sync_copy(acc.at[pl.ds(s * PTN, PTN)],
                        deg_out.at[c, pl.ds(s * PTN, PTN)])

    return deg_kernel


def _make_agg(nwin):
    assert nwin % NIB == 0 and nwin >= 2 * NIB

    @functools.partial(
        pl.kernel,
        out_type=jax.ShapeDtypeStruct((NC, NACC, D), jnp.float32),
        mesh=_mesh,
        scratch_types=[
            pltpu.VMEM((NIB, 2, WE), jnp.int32),
            pltpu.VMEM((NBUF, WE, D), jnp.float32),
            pltpu.VMEM((16, D), jnp.float32),
            pltpu.VMEM_SHARED((NACC, D), jnp.float32),
            [pltpu.SemaphoreType.DMA] * NIB,
            [pltpu.SemaphoreType.DMA] * NBUF,
            [pltpu.SemaphoreType.DMA] * NBUF,
        ],
    )
    def agg_kernel(m_hbm, idx_hbm, g_out,
                   iring, buf, zb, acc, isems, gsems, ssems):
        c = lax.axis_index("c")
        s = lax.axis_index("s")
        w = c * NS + s

        def fz(i, _):
            zb[i // 8, pl.ds((i % 8) * 16, 16)] = jnp.zeros((16,), jnp.float32)
            return _
        lax.fori_loop(0, 16 * 8, fz, None)

        base = s * PTN

        def zacc(k, _):
            pltpu.sync_copy(zb, acc.at[pl.ds(base + k * 16, 16)])
            return _
        lax.fori_loop(0, PTN // 16, zacc, None)
        plsc.subcore_barrier()

        def i_start(j, ib):
            pltpu.async_copy(idx_hbm.at[w, j], iring.at[ib], isems[ib])

        def i_wait(j, ib):
            pltpu.make_async_copy(idx_hbm.at[w, j], iring.at[ib],
                                  isems[ib]).wait()

        def g_start(j, ib, b):
            pltpu.async_copy(m_hbm.at[iring.at[ib, 0]], buf.at[b], gsems[b])

        def g_wait(j, ib, b):
            pltpu.make_async_copy(m_hbm.at[iring.at[ib, 0]], buf.at[b],
                                  gsems[b]).wait()

        def s_start(j, ib, b):
            pltpu.async_copy(buf.at[b], acc.at[iring.at[ib, 1]], ssems[b],
                             add=True)

        def s_wait(j, ib, b):
            pltpu.make_async_copy(buf.at[b], acc.at[iring.at[ib, 1]],
                                  ssems[b]).wait()

        i_start(0, 0)
        i_start(1, 1)
        i_wait(0, 0)
        g_start(0, 0, 0)
        i_wait(1, 1)
        g_start(1, 1, 1)
        i_start(2, 2)
        g_wait(0, 0, 0)
        s_start(0, 0, 0)
        s_wait(0, 0, 0)
        i_wait(2, 2)
        g_start(2, 2, 0)
        i_start(3, 3)
        g_wait(1, 1, 1)
        s_start(1, 1, 1)

        def body(t, _):
            j0 = 2 + t * NIB
            for k in range(NIB):
                j = j0 + k
                b = (2 + k) % NBUF
                ib = (2 + k) % NIB
                s_wait(j - 1, (ib - 1) % NIB, 1 - b)
                i_wait(j + 1, (ib + 1) % NIB)
                g_start(j + 1, (ib + 1) % NIB, 1 - b)
                i_start(j + 2, (ib + 2) % NIB)
                g_wait(j, ib, b)
                s_start(j, ib, b)
            return _
        lax.fori_loop(0, (nwin - 4) // NIB, body, None)

        j = nwin - 2
        b, ib = j % NBUF, j % NIB
        s_wait(j - 1, (ib - 1) % NIB, 1 - b)
        i_wait(j + 1, (ib + 1) % NIB)
        g_start(j + 1, (ib + 1) % NIB, 1 - b)
        g_wait(j, ib, b)
        s_start(j, ib, b)
        j = nwin - 1
        b, ib = j % NBUF, j % NIB
        s_wait(j - 1, (ib - 1) % NIB, 1 - b)
        g_wait(j, ib, b)
        s_start(j, ib, b)
        s_wait(j, ib, b)

        plsc.subcore_barrier()
        pltpu.sync_copy(acc.at[pl.ds(base, PTN)],
                        g_out.at[c, pl.ds(base, PTN)])

    return agg_kernel


BR = 400


def _prep_body(x_ref, w_ref, deg_ref, m_ref, dinv_ref):
    deg = deg_ref[0] + deg_ref[1]
    dv = jnp.where(deg > 0.0, lax.rsqrt(jnp.maximum(deg, 1e-12)), 0.0)
    dinv_ref[...] = dv
    h = jax.lax.dot(x_ref[...], w_ref[...],
                    precision=jax.lax.Precision.HIGHEST)
    m_ref[...] = h * dv


def _prep(x, W1, deg2):
    grid = N // BR
    return pl.pallas_call(
        _prep_body,
        grid=(grid,),
        in_specs=[
            pl.BlockSpec((BR, D), lambda i: (i, 0)),
            pl.BlockSpec((D, D), lambda i: (0, 0)),
            pl.BlockSpec((NC, BR, 1), lambda i: (0, i, 0)),
        ],
        out_specs=[
            pl.BlockSpec((BR, D), lambda i: (i, 0)),
            pl.BlockSpec((BR, 1), lambda i: (i, 0)),
        ],
        out_shape=[
            jax.ShapeDtypeStruct((N, D), jnp.float32),
            jax.ShapeDtypeStruct((N, 1), jnp.float32),
        ],
    )(x, W1, deg2)


def _mid_body(g_ref, dinv_ref, b_ref, w_ref, m_ref):
    dv = dinv_ref[...]
    agg = (g_ref[0] + g_ref[1]) * dv + b_ref[...]
    z = jnp.maximum(agg, 0.0)
    h = jax.lax.dot(z, w_ref[...], precision=jax.lax.Precision.HIGHEST)
    m_ref[...] = h * dv


def _mid(g, dinv, b, W):
    grid = N // BR
    return pl.pallas_call(
        _mid_body,
        grid=(grid,),
        in_specs=[
            pl.BlockSpec((NC, BR, D), lambda i: (0, i, 0)),
            pl.BlockSpec((BR, 1), lambda i: (i, 0)),
            pl.BlockSpec((1, D), lambda i: (0, 0)),
            pl.BlockSpec((D, D), lambda i: (0, 0)),
        ],
        out_specs=pl.BlockSpec((BR, D), lambda i: (i, 0)),
        out_shape=jax.ShapeDtypeStruct((N, D), jnp.float32),
    )(g, dinv, b.reshape(1, D), W)


def _final_body(g_ref, dinv_ref, b_ref, o_ref):
    dv = dinv_ref[...]
    o_ref[...] = (g_ref[0] + g_ref[1]) * dv + b_ref[...]


def _final(g, dinv, b):
    grid = N // BR
    return pl.pallas_call(
        _final_body,
        grid=(grid,),
        in_specs=[
            pl.BlockSpec((NC, BR, D), lambda i: (0, i, 0)),
            pl.BlockSpec((BR, 1), lambda i: (i, 0)),
            pl.BlockSpec((1, D), lambda i: (0, 0)),
        ],
        out_specs=pl.BlockSpec((BR, D), lambda i: (i, 0)),
        out_shape=jax.ShapeDtypeStruct((N, D), jnp.float32),
    )(g, dinv, b.reshape(1, D))


def kernel(x, adj_t, W1, b1, W2, b2, W3, b3):
    adj = adj_t.astype(jnp.int32)
    E = adj.shape[1]
    nwin = _nwin(E)
    epad = NW * WE * nwin
    pad = epad - E
    prange = jnp.arange(pad, dtype=jnp.int32)
    src = jnp.concatenate([adj[0], prange % N])
    dst = jnp.concatenate([adj[1], N + prange % NPAD_ROWS])
    idx_w = jnp.stack(
        [src.reshape(NW, nwin, WE), dst.reshape(NW, nwin, WE)], axis=2)

    deg2 = _make_deg(nwin)(idx_w)
    agg = _make_agg(nwin)
    m1, dinv = _prep(x, W1, deg2.reshape(NC, NACC, 1))
    g1 = agg(m1, idx_w)
    m2 = _mid(g1, dinv, b1, W2)
    g2 = agg(m2, idx_w)
    m3 = _mid(g2, dinv, b2, W3)
    g3 = agg(m3, idx_w)
    return _final(g3, dinv, b3)


# speedup vs baseline: 3.6968x; 1.0760x over previous
import functools

import jax
import jax.numpy as jnp
from jax import lax
from jax.experimental import pallas as pl
from jax.experimental.pallas import tpu as pltpu
from jax.experimental.pallas import tpu_sc as plsc

N = 10000
D = 128
NC = 2
NS = 16
NW = NC * NS
WE = 128
NACC = 10240
PTN = NACC // NS
NPAD_ROWS = NACC - N
NBUF = 2
NIB = 4

_mesh = plsc.VectorSubcoreMesh(core_axis_name="c", subcore_axis_name="s")


def _make_deg(nwin):
    @functools.partial(
        pl.kernel,
        out_type=jax.ShapeDtypeStruct((NC, NACC), jnp.float32),
        mesh=_mesh,
        scratch_types=[
            pltpu.VMEM((nwin, WE), jnp.int32),
            pltpu.VMEM((PTN,), jnp.float32),
            pltpu.VMEM((WE,), jnp.float32),
            pltpu.VMEM_SHARED((NACC,), jnp.float32),
        ],
    )
    def deg_kernel(dst_hbm, deg_out, dst_v, zv, ones_v, acc):
        c = lax.axis_index("c")
        s = lax.axis_index("s")
        w = c * NS + s

        def fz(i, _):
            zv[pl.ds(i * 16, 16)] = jnp.zeros((16,), jnp.float32)
            return _
        lax.fori_loop(0, PTN // 16, fz, None)

        def fo(i, _):
            ones_v[pl.ds(i * 16, 16)] = jnp.ones((16,), jnp.float32)
            return _
        lax.fori_loop(0, WE // 16, fo, None)

        pltpu.sync_copy(dst_hbm.at[w], dst_v)
        pltpu.sync_copy(zv, acc.at[pl.ds(s * PTN, PTN)])
        plsc.subcore_barrier()

        def body(j, _):
            pltpu.sync_copy(ones_v, acc.at[dst_v.at[j]], add=True)
            return _
        lax.fori_loop(0, nwin, body, None)

        plsc.subcore_barrier()
        pltpu.sync_copy(acc.at[pl.ds(s * PTN, PTN)],
                        deg_out.at[c, pl.ds(s * PTN, PTN)])

    return deg_kernel


GW = 8
NIG = 4


def _make_agg(nwin):
    assert nwin % GW == 0 and nwin >= 3 * GW
    ngrp = nwin // GW

    @functools.partial(
        pl.kernel,
        out_type=jax.ShapeDtypeStruct((NC, NACC, D), jnp.float32),
        mesh=_mesh,
        scratch_types=[
            pltpu.VMEM((NIG, GW, WE), jnp.int32),
            pltpu.VMEM((NIG, GW, WE), jnp.int32),
            pltpu.VMEM((NBUF, WE, D), jnp.float32),
            pltpu.VMEM((16, D), jnp.float32),
            pltpu.VMEM_SHARED((NACC, D), jnp.float32),
            [pltpu.SemaphoreType.DMA] * NIG,
            [pltpu.SemaphoreType.DMA] * NBUF,
            [pltpu.SemaphoreType.DMA] * NBUF,
        ],
    )
    def agg_kernel(m_hbm, src_hbm, dst_hbm, g_out,
                   sring, dring, buf, zb, acc, isems, gsems, ssems):
        c = lax.axis_index("c")
        s = lax.axis_index("s")
        w = c * NS + s

        def fz(i, _):
            zb[i // 8, pl.ds((i % 8) * 16, 16)] = jnp.zeros((16,), jnp.float32)
            return _
        lax.fori_loop(0, 16 * 8, fz, None)

        base = s * PTN

        def zacc(k, _):
            pltpu.sync_copy(zb, acc.at[pl.ds(base + k * 16, 16)])
            return _
        lax.fori_loop(0, PTN // 16, zacc, None)
        plsc.subcore_barrier()

        def i_start(g):
            ig = g % NIG
            pltpu.async_copy(src_hbm.at[w, pl.ds(g * GW, GW)],
                             sring.at[ig], isems[ig])
            pltpu.async_copy(dst_hbm.at[w, pl.ds(g * GW, GW)],
                             dring.at[ig], isems[ig])

        def i_wait(g):
            ig = g % NIG
            pltpu.make_async_copy(src_hbm.at[w, pl.ds(g * GW, GW)],
                                  sring.at[ig], isems[ig]).wait()
            pltpu.make_async_copy(dst_hbm.at[w, pl.ds(g * GW, GW)],
                                  dring.at[ig], isems[ig]).wait()

        def g_start(j):
            b = j % NBUF
            pltpu.async_copy(m_hbm.at[sring.at[(j // GW) % NIG, j % GW]],
                             buf.at[b], gsems[b])

        def g_wait(j):
            b = j % NBUF
            pltpu.make_async_copy(m_hbm.at[sring.at[(j // GW) % NIG, j % GW]],
                                  buf.at[b], gsems[b]).wait()

        def s_start(j):
            b = j % NBUF
            pltpu.async_copy(buf.at[b],
                             acc.at[dring.at[(j // GW) % NIG, j % GW]],
                             ssems[b], add=True)

        def s_wait(j):
            b = j % NBUF
            pltpu.make_async_copy(buf.at[b],
                                  acc.at[dring.at[(j // GW) % NIG, j % GW]],
                                  ssems[b]).wait()

        i_start(0)
        i_start(1)
        i_wait(0)
        g_start(0)
        for j in range(nwin):
            if j >= 1:
                s_wait(j - 1)
            if j % GW == 0 and j // GW + 2 < ngrp:
                i_start(j // GW + 2)
            if j % GW == GW - 1 and j + 1 < nwin:
                i_wait((j + 1) // GW)
            if j + 1 < nwin:
                g_start(j + 1)
            g_wait(j)
            s_start(j)
        s_wait(nwin - 1)

        plsc.subcore_barrier()
        pltpu.sync_copy(acc.at[pl.ds(base, PTN)],
                        g_out.at[c, pl.ds(base, PTN)])

    return agg_kernel


BR = 1000


def _prep_body(x_ref, w_ref, deg_ref, m_ref, dinv_ref):
    deg = deg_ref[0] + deg_ref[1]
    dv = jnp.where(deg > 0.0, lax.rsqrt(jnp.maximum(deg, 1e-12)), 0.0)
    dinv_ref[...] = dv
    m_ref[...] = jnp.dot(x_ref[...], w_ref[...]) * dv


def _prep(x, W1, deg2):
    grid = N // BR
    return pl.pallas_call(
        _prep_body,
        grid=(grid,),
        in_specs=[
            pl.BlockSpec((BR, D), lambda i: (i, 0)),
            pl.BlockSpec((D, D), lambda i: (0, 0)),
            pl.BlockSpec((NC, BR, 1), lambda i: (0, i, 0)),
        ],
        out_specs=[
            pl.BlockSpec((BR, D), lambda i: (i, 0)),
            pl.BlockSpec((BR, 1), lambda i: (i, 0)),
        ],
        out_shape=[
            jax.ShapeDtypeStruct((N, D), jnp.float32),
            jax.ShapeDtypeStruct((N, 1), jnp.float32),
        ],
    )(x, W1, deg2)


def _mid_body(g_ref, dinv_ref, b_ref, w_ref, m_ref):
    dv = dinv_ref[...]
    agg = (g_ref[0] + g_ref[1]) * dv + b_ref[...]
    z = jnp.maximum(agg, 0.0)
    m_ref[...] = jnp.dot(z, w_ref[...]) * dv


def _mid(g, dinv, b, W):
    grid = N // BR
    return pl.pallas_call(
        _mid_body,
        grid=(grid,),
        in_specs=[
            pl.BlockSpec((NC, BR, D), lambda i: (0, i, 0)),
            pl.BlockSpec((BR, 1), lambda i: (i, 0)),
            pl.BlockSpec((1, D), lambda i: (0, 0)),
            pl.BlockSpec((D, D), lambda i: (0, 0)),
        ],
        out_specs=pl.BlockSpec((BR, D), lambda i: (i, 0)),
        out_shape=jax.ShapeDtypeStruct((N, D), jnp.float32),
    )(g, dinv, b.reshape(1, D), W)


def _final_body(g_ref, dinv_ref, b_ref, o_ref):
    dv = dinv_ref[...]
    o_ref[...] = (g_ref[0] + g_ref[1]) * dv + b_ref[...]


def _final(g, dinv, b):
    grid = N // BR
    return pl.pallas_call(
        _final_body,
        grid=(grid,),
        in_specs=[
            pl.BlockSpec((NC, BR, D), lambda i: (0, i, 0)),
            pl.BlockSpec((BR, 1), lambda i: (i, 0)),
            pl.BlockSpec((1, D), lambda i: (0, 0)),
        ],
        out_specs=pl.BlockSpec((BR, D), lambda i: (i, 0)),
        out_shape=jax.ShapeDtypeStruct((N, D), jnp.float32),
    )(g, dinv, b.reshape(1, D))


def kernel(x, adj_t, W1, b1, W2, b2, W3, b3):
    adj = adj_t.astype(jnp.int32)
    E = adj.shape[1]
    chunk = NW * WE * GW
    nwin = ((E + chunk - 1) // chunk) * GW
    pad = NW * WE * nwin - E
    src, dst = adj[0], adj[1]
    if pad:
        prange = jnp.arange(pad, dtype=jnp.int32)
        src = jnp.concatenate([src, prange % N])
        dst = jnp.concatenate([dst, N + prange % NPAD_ROWS])
    src_w = src.reshape(NW, nwin, WE)
    dst_w = dst.reshape(NW, nwin, WE)

    deg2 = _make_deg(nwin)(dst_w)
    agg = _make_agg(nwin)
    m1, dinv = _prep(x, W1, deg2.reshape(NC, NACC, 1))
    g1 = agg(m1, src_w, dst_w)
    m2 = _mid(g1, dinv, b1, W2)
    g2 = agg(m2, src_w, dst_w)
    m3 = _mid(g2, dinv, b2, W3)
    g3 = agg(m3, src_w, dst_w)
    return _final(g3, dinv, b3)


# speedup vs baseline: 3.7238x; 1.0073x over previous
import functools

import jax
import jax.numpy as jnp
from jax import lax
from jax.experimental import pallas as pl
from jax.experimental.pallas import tpu as pltpu
from jax.experimental.pallas import tpu_sc as plsc

N = 10000
D = 128
NC = 2
NS = 16
NW = NC * NS
WE = 128
NACC = 10240
PTN = NACC // NS
NBUF = 2
NIB = 4

_mesh = plsc.VectorSubcoreMesh(core_axis_name="c", subcore_axis_name="s")


def _make_deg(nrow):
    rw, rem = nrow // NW, nrow % NW

    @functools.partial(
        pl.kernel,
        out_type=jax.ShapeDtypeStruct((NC, NACC), jnp.float32),
        mesh=_mesh,
        scratch_types=[
            pltpu.VMEM((NIB, WE), jnp.int32),
            pltpu.VMEM((PTN,), jnp.float32),
            pltpu.VMEM((WE,), jnp.float32),
            pltpu.VMEM_SHARED((NACC,), jnp.float32),
            [pltpu.SemaphoreType.DMA] * NIB,
        ],
    )
    def deg_kernel(dst_hbm, deg_out, dring, zv, ones_v, acc, isems):
        c = lax.axis_index("c")
        s = lax.axis_index("s")
        w = c * NS + s
        r0 = w * rw

        def fz(i, _):
            zv[pl.ds(i * 16, 16)] = jnp.zeros((16,), jnp.float32)
            return _
        lax.fori_loop(0, PTN // 16, fz, None)

        def fo(i, _):
            ones_v[pl.ds(i * 16, 16)] = jnp.ones((16,), jnp.float32)
            return _
        lax.fori_loop(0, WE // 16, fo, None)

        def i_start(j):
            ib = j % NIB
            pltpu.async_copy(dst_hbm.at[pl.ds((r0 + j) * WE, WE)],
                             dring.at[ib], isems[ib])

        def i_wait(j):
            ib = j % NIB
            pltpu.make_async_copy(dst_hbm.at[pl.ds((r0 + j) * WE, WE)],
                                  dring.at[ib], isems[ib]).wait()

        pltpu.sync_copy(zv, acc.at[pl.ds(s * PTN, PTN)])
        for j in range(min(3, rw)):
            i_start(j)
        plsc.subcore_barrier()

        for j in range(rw):
            i_wait(j)
            if j + 3 < rw:
                i_start(j + 3)
            pltpu.sync_copy(ones_v, acc.at[dring.at[j % NIB]], add=True)
        if rem:
            @pl.when(w < rem)
            def _():
                pltpu.sync_copy(
                    dst_hbm.at[pl.ds((NW * rw + w) * WE, WE)], dring.at[0])
                pltpu.sync_copy(ones_v, acc.at[dring.at[0]], add=True)

        plsc.subcore_barrier()
        pltpu.sync_copy(acc.at[pl.ds(s * PTN, PTN)],
                        deg_out.at[c, pl.ds(s * PTN, PTN)])

    return deg_kernel


def _make_agg(nrow):
    rw, rem = nrow // NW, nrow % NW
    assert rw >= 2 * NIB

    @functools.partial(
        pl.kernel,
        out_type=jax.ShapeDtypeStruct((NC, NACC, D), jnp.float32),
        mesh=_mesh,
        scratch_types=[
            pltpu.VMEM((NIB, WE), jnp.int32),
            pltpu.VMEM((NIB, WE), jnp.int32),
            pltpu.VMEM((NBUF, WE, D), jnp.float32),
            pltpu.VMEM((16, D), jnp.float32),
            pltpu.VMEM_SHARED((NACC, D), jnp.float32),
            [pltpu.SemaphoreType.DMA] * NIB,
            [pltpu.SemaphoreType.DMA] * NBUF,
            [pltpu.SemaphoreType.DMA] * NBUF,
        ],
    )
    def agg_kernel(m_hbm, src_hbm, dst_hbm, g_out,
                   sring, dring, buf, zb, acc, isems, gsems, ssems):
        c = lax.axis_index("c")
        s = lax.axis_index("s")
        w = c * NS + s
        r0 = w * rw

        def fz(i, _):
            zb[i // 8, pl.ds((i % 8) * 16, 16)] = jnp.zeros((16,), jnp.float32)
            return _
        lax.fori_loop(0, 16 * 8, fz, None)

        base = s * PTN

        def zacc(k, _):
            pltpu.sync_copy(zb, acc.at[pl.ds(base + k * 16, 16)])
            return _
        lax.fori_loop(0, PTN // 16, zacc, None)
        plsc.subcore_barrier()

        def i_start(j):
            ib = j % NIB
            pltpu.async_copy(src_hbm.at[pl.ds((r0 + j) * WE, WE)],
                             sring.at[ib], isems[ib])
            pltpu.async_copy(dst_hbm.at[pl.ds((r0 + j) * WE, WE)],
                             dring.at[ib], isems[ib])

        def i_wait(j):
            ib = j % NIB
            pltpu.make_async_copy(src_hbm.at[pl.ds((r0 + j) * WE, WE)],
                                  sring.at[ib], isems[ib]).wait()
            pltpu.make_async_copy(dst_hbm.at[pl.ds((r0 + j) * WE, WE)],
                                  dring.at[ib], isems[ib]).wait()

        def g_start(j):
            b = j % NBUF
            pltpu.async_copy(m_hbm.at[sring.at[j % NIB]], buf.at[b], gsems[b])

        def g_wait(j):
            b = j % NBUF
            pltpu.make_async_copy(m_hbm.at[sring.at[j % NIB]], buf.at[b],
                                  gsems[b]).wait()

        def s_start(j):
            b = j % NBUF
            pltpu.async_copy(buf.at[b], acc.at[dring.at[j % NIB]],
                             ssems[b], add=True)

        def s_wait(j):
            b = j % NBUF
            pltpu.make_async_copy(buf.at[b], acc.at[dring.at[j % NIB]],
                                  ssems[b]).wait()

        i_start(0)
        i_start(1)
        i_wait(0)
        g_start(0)
        for j in range(rw):
            if j >= 1:
                s_wait(j - 1)
            if j + 1 < rw:
                i_wait(j + 1)
                g_start(j + 1)
            if j + 2 < rw:
                i_start(j + 2)
            g_wait(j)
            s_start(j)
        s_wait(rw - 1)

        if rem:
            @pl.when(w < rem)
            def _():
                roff = (NW * rw + w) * WE
                pltpu.sync_copy(src_hbm.at[pl.ds(roff, WE)], sring.at[0])
                pltpu.sync_copy(dst_hbm.at[pl.ds(roff, WE)], dring.at[0])
                pltpu.sync_copy(m_hbm.at[sring.at[0]], buf.at[0])
                pltpu.sync_copy(buf.at[0], acc.at[dring.at[0]], add=True)

        plsc.subcore_barrier()
        pltpu.sync_copy(acc.at[pl.ds(base, PTN)],
                        g_out.at[c, pl.ds(base, PTN)])

    return agg_kernel


BR = 2000


def _prep_body(x_ref, w_ref, deg_ref, m_ref, dinv_ref):
    deg = deg_ref[0] + deg_ref[1]
    dv = jnp.where(deg > 0.0, lax.rsqrt(jnp.maximum(deg, 1e-12)), 0.0)
    dinv_ref[...] = dv
    m_ref[...] = jnp.dot(x_ref[...], w_ref[...]) * dv


def _prep(x, W1, deg2):
    grid = N // BR
    return pl.pallas_call(
        _prep_body,
        grid=(grid,),
        in_specs=[
            pl.BlockSpec((BR, D), lambda i: (i, 0)),
            pl.BlockSpec((D, D), lambda i: (0, 0)),
            pl.BlockSpec((NC, BR, 1), lambda i: (0, i, 0)),
        ],
        out_specs=[
            pl.BlockSpec((BR, D), lambda i: (i, 0)),
            pl.BlockSpec((BR, 1), lambda i: (i, 0)),
        ],
        out_shape=[
            jax.ShapeDtypeStruct((N, D), jnp.float32),
            jax.ShapeDtypeStruct((N, 1), jnp.float32),
        ],
    )(x, W1, deg2)


def _mid_body(g_ref, dinv_ref, b_ref, w_ref, m_ref):
    dv = dinv_ref[...]
    agg = (g_ref[0] + g_ref[1]) * dv + b_ref[...]
    z = jnp.maximum(agg, 0.0)
    m_ref[...] = jnp.dot(z, w_ref[...]) * dv


def _mid(g, dinv, b, W):
    grid = N // BR
    return pl.pallas_call(
        _mid_body,
        grid=(grid,),
        in_specs=[
            pl.BlockSpec((NC, BR, D), lambda i: (0, i, 0)),
            pl.BlockSpec((BR, 1), lambda i: (i, 0)),
            pl.BlockSpec((1, D), lambda i: (0, 0)),
            pl.BlockSpec((D, D), lambda i: (0, 0)),
        ],
        out_specs=pl.BlockSpec((BR, D), lambda i: (i, 0)),
        out_shape=jax.ShapeDtypeStruct((N, D), jnp.float32),
    )(g, dinv, b.reshape(1, D), W)


def _final_body(g_ref, dinv_ref, b_ref, o_ref):
    dv = dinv_ref[...]
    o_ref[...] = (g_ref[0] + g_ref[1]) * dv + b_ref[...]


def _final(g, dinv, b):
    grid = N // BR
    return pl.pallas_call(
        _final_body,
        grid=(grid,),
        in_specs=[
            pl.BlockSpec((NC, BR, D), lambda i: (0, i, 0)),
            pl.BlockSpec((BR, 1), lambda i: (i, 0)),
            pl.BlockSpec((1, D), lambda i: (0, 0)),
        ],
        out_specs=pl.BlockSpec((BR, D), lambda i: (i, 0)),
        out_shape=jax.ShapeDtypeStruct((N, D), jnp.float32),
    )(g, dinv, b.reshape(1, D))


def kernel(x, adj_t, W1, b1, W2, b2, W3, b3):
    adj = adj_t.astype(jnp.int32)
    E = adj.shape[1]
    src, dst = adj[0], adj[1]
    tail = (-E) % WE
    if tail:
        prange = jnp.arange(tail, dtype=jnp.int32)
        src = jnp.concatenate([src, prange % N])
        dst = jnp.concatenate([dst, N + prange % (NACC - N)])
    nrow = (E + tail) // WE

    deg2 = _make_deg(nrow)(dst)
    agg = _make_agg(nrow)
    m1, dinv = _prep(x, W1, deg2.reshape(NC, NACC, 1))
    g1 = agg(m1, src, dst)
    m2 = _mid(g1, dinv, b1, W2)
    g2 = agg(m2, src, dst)
    m3 = _mid(g2, dinv, b2, W3)
    g3 = agg(m3, src, dst)
    return _final(g3, dinv, b3)
